# initial kernel scaffold (unmeasured)
import jax
import jax.numpy as jnp
from jax import lax
from jax.experimental import pallas as pl
from jax.experimental.pallas import tpu as pltpu

M_CHUNK = 512


def kernel(A, B):
    m, k = A.shape
    k2, n = B.shape
    assert k == k2
    num_chunks = m // M_CHUNK

    def body(A_ref, B_hbm, out_ref, b_vmem, send_buf, comm_buf,
             local_sem, send_sems, recv_sems):
        i = pl.program_id(0)
        my_x = lax.axis_index("x")
        my_y = lax.axis_index("y")
        nbr = (my_x, 1 - my_y)

        @pl.when(i == 0)
        def _():
            barrier = pltpu.get_barrier_semaphore()
            pl.semaphore_signal(barrier, inc=1, device_id=nbr,
                                device_id_type=pl.DeviceIdType.MESH)
            pl.semaphore_wait(barrier, 1)
            cp = pltpu.make_async_copy(B_hbm, b_vmem, local_sem)
            cp.start()
            cp.wait()

        send_buf[...] = jnp.dot(A_ref[...], b_vmem[...],
                                preferred_element_type=jnp.float32)

        slot = lax.rem(i, 2)
        rdma = pltpu.make_async_remote_copy(
            src_ref=send_buf,
            dst_ref=comm_buf.at[slot],
            send_sem=send_sems.at[slot],
            recv_sem=recv_sems.at[slot],
            device_id=nbr,
            device_id_type=pl.DeviceIdType.MESH,
        )
        rdma.start()
        rdma.wait()

        out_ref[...] = send_buf[...] + comm_buf[slot]

    grid = (num_chunks,)
    return pl.pallas_call(
        body,
        grid=grid,
        out_shape=jax.ShapeDtypeStruct((m, n), jnp.float32),
        in_specs=[
            pl.BlockSpec((M_CHUNK, k), lambda i: (i, 0)),
            pl.BlockSpec(memory_space=pltpu.ANY),
        ],
        out_specs=pl.BlockSpec((M_CHUNK, n), lambda i: (i, 0)),
        scratch_shapes=[
            pltpu.VMEM((k, n), jnp.float32),
            pltpu.VMEM((M_CHUNK, n), jnp.float32),
            pltpu.VMEM((2, M_CHUNK, n), jnp.float32),
            pltpu.SemaphoreType.DMA,
            pltpu.SemaphoreType.DMA((2,)),
            pltpu.SemaphoreType.DMA((2,)),
        ],
        compiler_params=pltpu.CompilerParams(
            collective_id=0,
            dimension_semantics=("arbitrary",),
        ),
    )(A, B)


# baseline (device time: 1006103 ns/iter reference)
import jax
import jax.numpy as jnp
from jax import lax
from jax.experimental import pallas as pl
from jax.experimental.pallas import tpu as pltpu

M_CHUNK = 256
K_CHUNK = 512


def kernel(A, B):
    m, k = A.shape
    k2, n = B.shape
    assert k == k2
    num_m = m // M_CHUNK
    num_k = k // K_CHUNK

    def body(A_ref, B_ref, out_ref, send_buf, comm_buf, send_sems, recv_sems):
        i = pl.program_id(0)
        j = pl.program_id(1)
        my_x = lax.axis_index("x")
        my_y = lax.axis_index("y")
        nbr = (my_x, 1 - my_y)

        @pl.when((i == 0) & (j == 0))
        def _():
            barrier = pltpu.get_barrier_semaphore()
            pl.semaphore_signal(barrier, inc=1, device_id=nbr,
                                device_id_type=pl.DeviceIdType.MESH)
            pl.semaphore_wait(barrier, 1)

        partial = jnp.dot(A_ref[...], B_ref[...],
                          preferred_element_type=jnp.float32)

        @pl.when(j == 0)
        def _():
            send_buf[...] = partial

        @pl.when(j > 0)
        def _():
            send_buf[...] += partial

        @pl.when(j == num_k - 1)
        def _():
            slot = lax.rem(i, 2)
            rdma = pltpu.make_async_remote_copy(
                src_ref=send_buf,
                dst_ref=comm_buf.at[slot],
                send_sem=send_sems.at[slot],
                recv_sem=recv_sems.at[slot],
                device_id=nbr,
                device_id_type=pl.DeviceIdType.MESH,
            )
            rdma.start()
            rdma.wait()
            out_ref[...] = send_buf[...] + comm_buf[slot]

    grid = (num_m, num_k)
    return pl.pallas_call(
        body,
        grid=grid,
        out_shape=jax.ShapeDtypeStruct((m, n), jnp.float32),
        in_specs=[
            pl.BlockSpec((M_CHUNK, K_CHUNK), lambda i, j: (i, j)),
            pl.BlockSpec((K_CHUNK, n), lambda i, j: (j, 0)),
        ],
        out_specs=pl.BlockSpec((M_CHUNK, n), lambda i, j: (i, 0)),
        scratch_shapes=[
            pltpu.VMEM((M_CHUNK, n), jnp.float32),
            pltpu.VMEM((2, M_CHUNK, n), jnp.float32),
            pltpu.SemaphoreType.DMA((2,)),
            pltpu.SemaphoreType.DMA((2,)),
        ],
        compiler_params=pltpu.CompilerParams(
            collective_id=0,
            dimension_semantics=("arbitrary", "arbitrary"),
            vmem_limit_bytes=60 * 1024 * 1024,
        ),
    )(A, B)


# device time: 782903 ns/iter; 1.2851x vs baseline; 1.2851x over previous
import jax
import jax.numpy as jnp
from jax import lax
from jax.experimental import pallas as pl
from jax.experimental.pallas import tpu as pltpu

M_CHUNK = 256
K_CHUNK = 512


def kernel(A, B):
    m, k = A.shape
    k2, n = B.shape
    assert k == k2
    num_m = m // M_CHUNK
    num_k = k // K_CHUNK

    def body(A_ref, B_ref, out_ref, send_buf, comm_buf, send_sems, recv_sems):
        i = pl.program_id(0)
        j = pl.program_id(1)
        my_x = lax.axis_index("x")
        my_y = lax.axis_index("y")
        nbr = (my_x, 1 - my_y)

        @pl.when((i == 0) & (j == 0))
        def _():
            barrier = pltpu.get_barrier_semaphore()
            pl.semaphore_signal(barrier, inc=1, device_id=nbr,
                                device_id_type=pl.DeviceIdType.MESH)
            pl.semaphore_wait(barrier, 1)

        acc = lax.rem(i, 2)
        partial = jnp.dot(A_ref[...], B_ref[...],
                          preferred_element_type=jnp.float32)

        @pl.when(j == 0)
        def _():
            send_buf[acc] = partial

        @pl.when(j > 0)
        def _():
            send_buf[acc] += partial

        @pl.when(j == num_k - 1)
        def _():
            @pl.when(i < num_m)
            def _():
                slot = lax.rem(i, 4)
                rdma = pltpu.make_async_remote_copy(
                    src_ref=send_buf.at[acc],
                    dst_ref=comm_buf.at[slot],
                    send_sem=send_sems.at[slot],
                    recv_sem=recv_sems.at[slot],
                    device_id=nbr,
                    device_id_type=pl.DeviceIdType.MESH,
                )
                rdma.start()

            @pl.when(i > 0)
            def _():
                prev = i - 1
                pacc = lax.rem(prev, 2)
                pslot = lax.rem(prev, 4)
                prev_rdma = pltpu.make_async_remote_copy(
                    src_ref=send_buf.at[pacc],
                    dst_ref=comm_buf.at[pslot],
                    send_sem=send_sems.at[pslot],
                    recv_sem=recv_sems.at[pslot],
                    device_id=nbr,
                    device_id_type=pl.DeviceIdType.MESH,
                )
                prev_rdma.wait()
                out_ref[...] = send_buf[pacc] + comm_buf[pslot]

    grid = (num_m + 1, num_k)
    last_m = num_m - 1
    return pl.pallas_call(
        body,
        grid=grid,
        out_shape=jax.ShapeDtypeStruct((m, n), jnp.float32),
        in_specs=[
            pl.BlockSpec((M_CHUNK, K_CHUNK),
                         lambda i, j: (jnp.minimum(i, last_m), j)),
            pl.BlockSpec((K_CHUNK, n), lambda i, j: (j, 0)),
        ],
        out_specs=pl.BlockSpec((M_CHUNK, n),
                               lambda i, j: (jnp.maximum(i - 1, 0), 0)),
        scratch_shapes=[
            pltpu.VMEM((2, M_CHUNK, n), jnp.float32),
            pltpu.VMEM((4, M_CHUNK, n), jnp.float32),
            pltpu.SemaphoreType.DMA((4,)),
            pltpu.SemaphoreType.DMA((4,)),
        ],
        compiler_params=pltpu.CompilerParams(
            collective_id=0,
            dimension_semantics=("arbitrary", "arbitrary"),
            vmem_limit_bytes=60 * 1024 * 1024,
        ),
    )(A, B)


# device time: 423207 ns/iter; 2.3773x vs baseline; 1.8499x over previous
import jax
import jax.numpy as jnp
from jax import lax
from jax.experimental import pallas as pl
from jax.experimental.pallas import tpu as pltpu

M_CHUNK = 256
K_CHUNK = 512


def kernel(A, B):
    m, k = A.shape
    k2, n = B.shape
    assert k == k2
    num_m = m // M_CHUNK
    num_k = k // K_CHUNK

    def body(A_ref, B_ref, out_ref, send_buf, wire_buf, comm_buf,
             send_sems, recv_sems):
        i = pl.program_id(0)
        j = pl.program_id(1)
        my_x = lax.axis_index("x")
        my_y = lax.axis_index("y")
        nbr = (my_x, 1 - my_y)

        @pl.when((i == 0) & (j == 0))
        def _():
            barrier = pltpu.get_barrier_semaphore()
            pl.semaphore_signal(barrier, inc=1, device_id=nbr,
                                device_id_type=pl.DeviceIdType.MESH)
            pl.semaphore_wait(barrier, 1)

        acc = lax.rem(i, 2)
        partial = jnp.dot(A_ref[...], B_ref[...],
                          preferred_element_type=jnp.float32)

        @pl.when(j == 0)
        def _():
            send_buf[acc] = partial

        @pl.when(j > 0)
        def _():
            send_buf[acc] += partial

        @pl.when(j == num_k - 1)
        def _():
            @pl.when(i < num_m)
            def _():
                slot = lax.rem(i, 4)
                wire_buf[acc] = send_buf[acc].astype(jnp.bfloat16)
                rdma = pltpu.make_async_remote_copy(
                    src_ref=wire_buf.at[acc],
                    dst_ref=comm_buf.at[slot],
                    send_sem=send_sems.at[slot],
                    recv_sem=recv_sems.at[slot],
                    device_id=nbr,
                    device_id_type=pl.DeviceIdType.MESH,
                )
                rdma.start()

            @pl.when(i > 0)
            def _():
                prev = i - 1
                pacc = lax.rem(prev, 2)
                pslot = lax.rem(prev, 4)
                prev_rdma = pltpu.make_async_remote_copy(
                    src_ref=wire_buf.at[pacc],
                    dst_ref=comm_buf.at[pslot],
                    send_sem=send_sems.at[pslot],
                    recv_sem=recv_sems.at[pslot],
                    device_id=nbr,
                    device_id_type=pl.DeviceIdType.MESH,
                )
                prev_rdma.wait()
                out_ref[...] = send_buf[pacc] + comm_buf[pslot].astype(
                    jnp.float32)

    grid = (num_m + 1, num_k)
    last_m = num_m - 1
    return pl.pallas_call(
        body,
        grid=grid,
        out_shape=jax.ShapeDtypeStruct((m, n), jnp.float32),
        in_specs=[
            pl.BlockSpec((M_CHUNK, K_CHUNK),
                         lambda i, j: (jnp.minimum(i, last_m), j)),
            pl.BlockSpec((K_CHUNK, n), lambda i, j: (j, 0)),
        ],
        out_specs=pl.BlockSpec((M_CHUNK, n),
                               lambda i, j: (jnp.maximum(i - 1, 0), 0)),
        scratch_shapes=[
            pltpu.VMEM((2, M_CHUNK, n), jnp.float32),
            pltpu.VMEM((2, M_CHUNK, n), jnp.bfloat16),
            pltpu.VMEM((4, M_CHUNK, n), jnp.bfloat16),
            pltpu.SemaphoreType.DMA((4,)),
            pltpu.SemaphoreType.DMA((4,)),
        ],
        compiler_params=pltpu.CompilerParams(
            collective_id=0,
            dimension_semantics=("arbitrary", "arbitrary"),
            vmem_limit_bytes=60 * 1024 * 1024,
        ),
    )(A, B)
